# Initial kernel scaffold; baseline (speedup 1.0000x reference)
import jax
import jax.numpy as jnp
from jax.experimental import pallas as pl


def kernel(x, edge_index, W_self0, W_neigh0, b0, W_self1, W_neigh1, b1, R_W0, R_b0, R_W1, R_b1, R_W2, R_b2):
    raise NotImplementedError("write your pallas kernel here")


# speedup vs baseline: 2.6569x; 2.6569x over previous
import functools

import jax
import jax.numpy as jnp
from jax import lax
from jax.experimental import pallas as pl
from jax.experimental.pallas import tpu as pltpu
from jax.experimental.pallas import tpu_sc as plsc

N = 10000
NP = 10240
E = 160000
TILES = 16
CB = 128
NCHUNK = 79
EP = TILES * NCHUNK * CB
ROWS_PT = NP // TILES
CH = 64
BM = 512


def _fill(ref, rows, cols, val):
    v = jnp.full((16,), val, jnp.float32)

    def body(r, _):
        for c in range(cols // 16):
            ref[r, pl.ds(c * 16, 16)] = v
        return 0

    lax.fori_loop(0, rows, body, 0)


def _make_agg(nc, with_deg):
    out_type = [jax.ShapeDtypeStruct((NP, CH), jnp.float32) for _ in range(2 * nc)]
    if with_deg:
        out_type.append(jax.ShapeDtypeStruct((TILES, NP), jnp.float32))

    scratch = [
        pltpu.VMEM((NCHUNK, CB), jnp.int32),
        pltpu.VMEM((NCHUNK, CB), jnp.int32),
        pltpu.VMEM((CB, CH), jnp.float32),
        pltpu.VMEM((CB, CH), jnp.float32),
        pltpu.VMEM_SHARED((NP, CH), jnp.float32),
    ]
    if with_deg:
        scratch.append(pltpu.VMEM((NP,), jnp.float32))
    scratch.append(pltpu.SemaphoreType.DMA)

    mesh = plsc.VectorSubcoreMesh(core_axis_name="c", subcore_axis_name="s")

    @functools.partial(pl.kernel, out_type=out_type, mesh=mesh,
                       scratch_types=scratch,
                       compiler_params=pltpu.CompilerParams(
                           needs_layout_passes=False,
                           use_tc_tiling_on_sc=False))
    def agg(src_hbm, dst_hbm, *rest):
        tables = rest[:2 * nc]
        outs = rest[2 * nc:4 * nc]
        k = 4 * nc
        if with_deg:
            deg_out = rest[k]; k += 1
        src_v = rest[k]; dst_v = rest[k + 1]
        rows_v = rest[k + 2]; zrow_v = rest[k + 3]
        acc = rest[k + 4]; k += 5
        if with_deg:
            hist_v = rest[k]; k += 1
        sem = rest[k]

        core = lax.axis_index("c")
        sid = lax.axis_index("s")
        row0 = sid * ROWS_PT

        pltpu.sync_copy(src_hbm.at[sid], src_v)
        pltpu.sync_copy(dst_hbm.at[sid], dst_v)

        _fill(zrow_v, CB, CH, 0.0)
        if with_deg:
            @pl.when(core == 0)
            def _():
                zv = jnp.zeros((16,), jnp.float32)

                def zb(rr, _):
                    hist_v[pl.ds(rr * 16, 16)] = zv
                    return 0
                lax.fori_loop(0, NP // 16, zb, 0)

                ones16 = jnp.ones((16,), jnp.float32)

                def db(j, _):
                    for c16 in range(CB // 16):
                        idx = dst_v[j, pl.ds(c16 * 16, 16)]
                        plsc.addupdate_scatter(hist_v, [idx], ones16)
                    return 0
                lax.fori_loop(0, NCHUNK, db, 0)
                pltpu.sync_copy(hist_v, deg_out.at[sid])

        for i in range(nc):
            for b in range(ROWS_PT // CB):
                pltpu.sync_copy(zrow_v, acc.at[pl.ds(row0 + b * CB, CB)])
            plsc.subcore_barrier()

            def edge_body(j, _, i=i):
                for c in range(2):
                    @pl.when(core == c)
                    def _(c=c):
                        tab = tables[c * nc + i]
                        pltpu.async_copy(tab.at[src_v.at[j]], rows_v, sem).wait()
                        pltpu.sync_copy(rows_v, acc.at[dst_v.at[j]], add=True)
                return 0

            lax.fori_loop(0, NCHUNK, edge_body, 0)
            plsc.subcore_barrier()

            for c in range(2):
                @pl.when(core == c)
                def _(c=c, i=i):
                    pltpu.sync_copy(acc.at[pl.ds(row0, ROWS_PT)],
                                    outs[c * nc + i].at[pl.ds(row0, ROWS_PT)])

    return agg


_agg1 = _make_agg(nc=2, with_deg=True)
_agg2 = _make_agg(nc=4, with_deg=False)


def _tc1_body(x_ref, a0_ref, a1_ref, a2_ref, a3_ref, d_ref, ws_ref, wn_ref,
              b_ref, o_ref):
    deg = jnp.sum(d_ref[...], axis=0).reshape(BM, 1)
    r = 1.0 / jnp.maximum(deg, 1.0)
    agg = jnp.concatenate(
        [a0_ref[...], a1_ref[...], a2_ref[...], a3_ref[...]], axis=1) * r
    h = jnp.dot(x_ref[...], ws_ref[...], preferred_element_type=jnp.float32)
    h = h + jnp.dot(agg, wn_ref[...], preferred_element_type=jnp.float32)
    h = h + b_ref[...]
    o_ref[...] = jnp.maximum(h, 0.0)


_tc1 = pl.pallas_call(
    _tc1_body,
    grid=(NP // BM,),
    in_specs=[
        pl.BlockSpec((BM, 256), lambda m: (m, 0)),
        pl.BlockSpec((BM, CH), lambda m: (m, 0)),
        pl.BlockSpec((BM, CH), lambda m: (m, 0)),
        pl.BlockSpec((BM, CH), lambda m: (m, 0)),
        pl.BlockSpec((BM, CH), lambda m: (m, 0)),
        pl.BlockSpec((TILES, BM), lambda m: (0, m)),
        pl.BlockSpec((256, 512), lambda m: (0, 0)),
        pl.BlockSpec((256, 512), lambda m: (0, 0)),
        pl.BlockSpec((1, 512), lambda m: (0, 0)),
    ],
    out_specs=pl.BlockSpec((BM, 512), lambda m: (m, 0)),
    out_shape=jax.ShapeDtypeStruct((NP, 512), jnp.float32),
)


def _tc2_body(h_ref, g0_ref, g1_ref, g2_ref, g3_ref, g4_ref, g5_ref, g6_ref,
              g7_ref, d_ref, ws_ref, wn_ref,
              b1_ref, rw0_ref, rb0_ref, rw1_ref, rb1_ref, rw2_ref, rb2_ref,
              o_ref):
    deg = jnp.sum(d_ref[...], axis=0).reshape(BM, 1)
    r = 1.0 / jnp.maximum(deg, 1.0)
    neigh = jnp.concatenate(
        [g0_ref[...], g1_ref[...], g2_ref[...], g3_ref[...],
         g4_ref[...], g5_ref[...], g6_ref[...], g7_ref[...]], axis=1) * r
    h2 = jnp.dot(h_ref[...], ws_ref[...], preferred_element_type=jnp.float32)
    h2 = h2 + jnp.dot(neigh, wn_ref[...], preferred_element_type=jnp.float32)
    h2 = jnp.maximum(h2 + b1_ref[...], 0.0)
    t = jnp.maximum(
        jnp.dot(h2, rw0_ref[...], preferred_element_type=jnp.float32)
        + rb0_ref[...], 0.0)
    t = jnp.maximum(
        jnp.dot(t, rw1_ref[...], preferred_element_type=jnp.float32)
        + rb1_ref[...], 0.0)
    o_ref[...] = (jnp.dot(t, rw2_ref[...], preferred_element_type=jnp.float32)
                  + rb2_ref[...])


_tc2 = pl.pallas_call(
    _tc2_body,
    grid=(NP // BM,),
    in_specs=[
        pl.BlockSpec((BM, 512), lambda m: (m, 0)),
        pl.BlockSpec((BM, CH), lambda m: (m, 0)),
        pl.BlockSpec((BM, CH), lambda m: (m, 0)),
        pl.BlockSpec((BM, CH), lambda m: (m, 0)),
        pl.BlockSpec((BM, CH), lambda m: (m, 0)),
        pl.BlockSpec((BM, CH), lambda m: (m, 0)),
        pl.BlockSpec((BM, CH), lambda m: (m, 0)),
        pl.BlockSpec((BM, CH), lambda m: (m, 0)),
        pl.BlockSpec((BM, CH), lambda m: (m, 0)),
        pl.BlockSpec((TILES, BM), lambda m: (0, m)),
        pl.BlockSpec((512, 512), lambda m: (0, 0)),
        pl.BlockSpec((512, 512), lambda m: (0, 0)),
        pl.BlockSpec((1, 512), lambda m: (0, 0)),
        pl.BlockSpec((512, 512), lambda m: (0, 0)),
        pl.BlockSpec((1, 512), lambda m: (0, 0)),
        pl.BlockSpec((512, 256), lambda m: (0, 0)),
        pl.BlockSpec((1, 256), lambda m: (0, 0)),
        pl.BlockSpec((256, 1), lambda m: (0, 0)),
        pl.BlockSpec((1, 1), lambda m: (0, 0)),
    ],
    out_specs=pl.BlockSpec((BM, 1), lambda m: (m, 0)),
    out_shape=jax.ShapeDtypeStruct((NP, 1), jnp.float32),
)


def kernel(x, edge_index, W_self0, W_neigh0, b0, W_self1, W_neigh1, b1,
           R_W0, R_b0, R_W1, R_b1, R_W2, R_b2):
    f32 = jnp.float32
    src = edge_index[0]
    dst = edge_index[1]
    padv = jnp.full((EP - E,), NP - 1, jnp.int32)
    src_p = jnp.concatenate([src, padv]).reshape(TILES, NCHUNK, CB)
    dst_p = jnp.concatenate([dst, padv]).reshape(TILES, NCHUNK, CB)
    xp = jnp.pad(x, ((0, NP - N), (0, 0)))

    xs = [lax.slice(xp, (0, c * CH), (NP, (c + 1) * CH)) for c in range(4)]
    a0, a1, a2, a3, degp = _agg1(src_p, dst_p, *xs)
    h = _tc1(xp, a0, a1, a2, a3, degp, W_self0, W_neigh0,
             b0.reshape(1, -1))
    g = [lax.slice(h, (0, c * CH), (NP, (c + 1) * CH)) for c in range(8)]
    gs = _agg2(src_p, dst_p, *g)
    out = _tc2(h, *gs, degp, W_self1, W_neigh1,
               b1.reshape(1, -1), R_W0, R_b0.reshape(1, -1),
               R_W1, R_b1.reshape(1, -1), R_W2, R_b2.reshape(1, -1))
    return out[:N]


# speedup vs baseline: 3.5350x; 1.3305x over previous
import functools

import jax
import jax.numpy as jnp
from jax import lax
from jax.experimental import pallas as pl
from jax.experimental.pallas import tpu as pltpu
from jax.experimental.pallas import tpu_sc as plsc

N = 10000
NP = 10240
E = 160000
TILES = 16
CB = 128
NCHUNK = 79
EP = TILES * NCHUNK * CB
ROWS_PT = NP // TILES
CH = 64
BM = 512


def _fill(ref, rows, cols, val):
    v = jnp.full((16,), val, jnp.float32)

    def body(r, _):
        for c in range(cols // 16):
            ref[r, pl.ds(c * 16, 16)] = v
        return 0

    lax.fori_loop(0, rows, body, 0)


def _make_agg(nc, with_deg):
    out_type = [jax.ShapeDtypeStruct((NP, CH), jnp.float32) for _ in range(2 * nc)]
    if with_deg:
        out_type.append(jax.ShapeDtypeStruct((TILES, NP), jnp.float32))

    scratch = [
        pltpu.VMEM((NCHUNK, CB), jnp.int32),
        pltpu.VMEM((NCHUNK, CB), jnp.int32),
        pltpu.VMEM((2, CB, CH), jnp.float32),
        pltpu.VMEM((CB, CH), jnp.float32),
        pltpu.VMEM_SHARED((NP, CH), jnp.float32),
    ]
    if with_deg:
        scratch.append(pltpu.VMEM((NP,), jnp.float32))
    scratch.append(pltpu.SemaphoreType.DMA)
    scratch.append(pltpu.SemaphoreType.DMA)

    mesh = plsc.VectorSubcoreMesh(core_axis_name="c", subcore_axis_name="s")

    @functools.partial(pl.kernel, out_type=out_type, mesh=mesh,
                       scratch_types=scratch,
                       compiler_params=pltpu.CompilerParams(
                           needs_layout_passes=False,
                           use_tc_tiling_on_sc=False))
    def agg(src_hbm, dst_hbm, *rest):
        tables = rest[:2 * nc]
        outs = rest[2 * nc:4 * nc]
        k = 4 * nc
        if with_deg:
            deg_out = rest[k]; k += 1
        src_v = rest[k]; dst_v = rest[k + 1]
        rows_v = rest[k + 2]; zrow_v = rest[k + 3]
        acc = rest[k + 4]; k += 5
        if with_deg:
            hist_v = rest[k]; k += 1
        sems = (rest[k], rest[k + 1])

        core = lax.axis_index("c")
        sid = lax.axis_index("s")
        row0 = sid * ROWS_PT

        pltpu.sync_copy(src_hbm.at[sid], src_v)
        pltpu.sync_copy(dst_hbm.at[sid], dst_v)

        _fill(zrow_v, CB, CH, 0.0)
        if with_deg:
            @pl.when(core == 0)
            def _():
                zv = jnp.zeros((16,), jnp.float32)

                def zb(rr, _):
                    hist_v[pl.ds(rr * 16, 16)] = zv
                    return 0
                lax.fori_loop(0, NP // 16, zb, 0)

                ones16 = jnp.ones((16,), jnp.float32)

                def db(j, _):
                    for c16 in range(CB // 16):
                        idx = dst_v[j, pl.ds(c16 * 16, 16)]
                        plsc.addupdate_scatter(hist_v, [idx], ones16)
                    return 0
                lax.fori_loop(0, NCHUNK, db, 0)
                pltpu.sync_copy(hist_v, deg_out.at[sid])

        for i in range(nc):
            for b in range(ROWS_PT // CB):
                pltpu.sync_copy(zrow_v, acc.at[pl.ds(row0 + b * CB, CB)])
            plsc.subcore_barrier()

            for c in range(2):
                @pl.when(core == c)
                def _(c=c, i=i):
                    tab = tables[c * nc + i]

                    def g_issue(j, p):
                        pltpu.async_copy(tab.at[src_v.at[j]], rows_v.at[p],
                                         sems[p])

                    def g_wait(j, p):
                        pltpu.make_async_copy(tab.at[src_v.at[j]],
                                              rows_v.at[p], sems[p]).wait()

                    def s_add(j, p):
                        pltpu.sync_copy(rows_v.at[p], acc.at[dst_v.at[j]],
                                        add=True)

                    g_issue(0, 0)

                    def edge_body(t, _):
                        j0 = 2 * t
                        g_issue(j0 + 1, 1)
                        g_wait(j0, 0)
                        s_add(j0, 0)
                        g_issue(j0 + 2, 0)
                        g_wait(j0 + 1, 1)
                        s_add(j0 + 1, 1)
                        return 0

                    lax.fori_loop(0, (NCHUNK - 1) // 2, edge_body, 0)
                    g_wait(NCHUNK - 1, 0)
                    s_add(NCHUNK - 1, 0)
            plsc.subcore_barrier()

            for c in range(2):
                @pl.when(core == c)
                def _(c=c, i=i):
                    pltpu.sync_copy(acc.at[pl.ds(row0, ROWS_PT)],
                                    outs[c * nc + i].at[pl.ds(row0, ROWS_PT)])

    return agg


_agg1 = _make_agg(nc=2, with_deg=True)
_agg2 = _make_agg(nc=4, with_deg=False)


def _tc1_body(x_ref, a0_ref, a1_ref, a2_ref, a3_ref, d_ref, ws_ref, wn_ref,
              b_ref, o_ref):
    deg = jnp.sum(d_ref[...], axis=0).reshape(BM, 1)
    r = 1.0 / jnp.maximum(deg, 1.0)
    agg = jnp.concatenate(
        [a0_ref[...], a1_ref[...], a2_ref[...], a3_ref[...]], axis=1) * r
    h = jnp.dot(x_ref[...], ws_ref[...], preferred_element_type=jnp.float32)
    h = h + jnp.dot(agg, wn_ref[...], preferred_element_type=jnp.float32)
    h = h + b_ref[...]
    o_ref[...] = jnp.maximum(h, 0.0)


_tc1 = pl.pallas_call(
    _tc1_body,
    grid=(NP // BM,),
    in_specs=[
        pl.BlockSpec((BM, 256), lambda m: (m, 0)),
        pl.BlockSpec((BM, CH), lambda m: (m, 0)),
        pl.BlockSpec((BM, CH), lambda m: (m, 0)),
        pl.BlockSpec((BM, CH), lambda m: (m, 0)),
        pl.BlockSpec((BM, CH), lambda m: (m, 0)),
        pl.BlockSpec((TILES, BM), lambda m: (0, m)),
        pl.BlockSpec((256, 512), lambda m: (0, 0)),
        pl.BlockSpec((256, 512), lambda m: (0, 0)),
        pl.BlockSpec((1, 512), lambda m: (0, 0)),
    ],
    out_specs=pl.BlockSpec((BM, 512), lambda m: (m, 0)),
    out_shape=jax.ShapeDtypeStruct((NP, 512), jnp.float32),
)


def _tc2_body(h_ref, g0_ref, g1_ref, g2_ref, g3_ref, g4_ref, g5_ref, g6_ref,
              g7_ref, d_ref, ws_ref, wn_ref,
              b1_ref, rw0_ref, rb0_ref, rw1_ref, rb1_ref, rw2_ref, rb2_ref,
              o_ref):
    deg = jnp.sum(d_ref[...], axis=0).reshape(BM, 1)
    r = 1.0 / jnp.maximum(deg, 1.0)
    neigh = jnp.concatenate(
        [g0_ref[...], g1_ref[...], g2_ref[...], g3_ref[...],
         g4_ref[...], g5_ref[...], g6_ref[...], g7_ref[...]], axis=1) * r
    h2 = jnp.dot(h_ref[...], ws_ref[...], preferred_element_type=jnp.float32)
    h2 = h2 + jnp.dot(neigh, wn_ref[...], preferred_element_type=jnp.float32)
    h2 = jnp.maximum(h2 + b1_ref[...], 0.0)
    t = jnp.maximum(
        jnp.dot(h2, rw0_ref[...], preferred_element_type=jnp.float32)
        + rb0_ref[...], 0.0)
    t = jnp.maximum(
        jnp.dot(t, rw1_ref[...], preferred_element_type=jnp.float32)
        + rb1_ref[...], 0.0)
    o_ref[...] = (jnp.dot(t, rw2_ref[...], preferred_element_type=jnp.float32)
                  + rb2_ref[...])


_tc2 = pl.pallas_call(
    _tc2_body,
    grid=(NP // BM,),
    in_specs=[
        pl.BlockSpec((BM, 512), lambda m: (m, 0)),
        pl.BlockSpec((BM, CH), lambda m: (m, 0)),
        pl.BlockSpec((BM, CH), lambda m: (m, 0)),
        pl.BlockSpec((BM, CH), lambda m: (m, 0)),
        pl.BlockSpec((BM, CH), lambda m: (m, 0)),
        pl.BlockSpec((BM, CH), lambda m: (m, 0)),
        pl.BlockSpec((BM, CH), lambda m: (m, 0)),
        pl.BlockSpec((BM, CH), lambda m: (m, 0)),
        pl.BlockSpec((BM, CH), lambda m: (m, 0)),
        pl.BlockSpec((TILES, BM), lambda m: (0, m)),
        pl.BlockSpec((512, 512), lambda m: (0, 0)),
        pl.BlockSpec((512, 512), lambda m: (0, 0)),
        pl.BlockSpec((1, 512), lambda m: (0, 0)),
        pl.BlockSpec((512, 512), lambda m: (0, 0)),
        pl.BlockSpec((1, 512), lambda m: (0, 0)),
        pl.BlockSpec((512, 256), lambda m: (0, 0)),
        pl.BlockSpec((1, 256), lambda m: (0, 0)),
        pl.BlockSpec((256, 1), lambda m: (0, 0)),
        pl.BlockSpec((1, 1), lambda m: (0, 0)),
    ],
    out_specs=pl.BlockSpec((BM, 1), lambda m: (m, 0)),
    out_shape=jax.ShapeDtypeStruct((NP, 1), jnp.float32),
)


def kernel(x, edge_index, W_self0, W_neigh0, b0, W_self1, W_neigh1, b1,
           R_W0, R_b0, R_W1, R_b1, R_W2, R_b2):
    f32 = jnp.float32
    src = edge_index[0]
    dst = edge_index[1]
    padv = jnp.full((EP - E,), NP - 1, jnp.int32)
    src_p = jnp.concatenate([src, padv]).reshape(TILES, NCHUNK, CB)
    dst_p = jnp.concatenate([dst, padv]).reshape(TILES, NCHUNK, CB)
    xp = jnp.pad(x, ((0, NP - N), (0, 0)))

    xs = [lax.slice(xp, (0, c * CH), (NP, (c + 1) * CH)) for c in range(4)]
    a0, a1, a2, a3, degp = _agg1(src_p, dst_p, *xs)
    h = _tc1(xp, a0, a1, a2, a3, degp, W_self0, W_neigh0,
             b0.reshape(1, -1))
    g = [lax.slice(h, (0, c * CH), (NP, (c + 1) * CH)) for c in range(8)]
    gs = _agg2(src_p, dst_p, *g)
    out = _tc2(h, *gs, degp, W_self1, W_neigh1,
               b1.reshape(1, -1), R_W0, R_b0.reshape(1, -1),
               R_W1, R_b1.reshape(1, -1), R_W2, R_b2.reshape(1, -1))
    return out[:N]


# speedup vs baseline: 4.0702x; 1.1514x over previous
import functools

import jax
import jax.numpy as jnp
from jax import lax
from jax.experimental import pallas as pl
from jax.experimental.pallas import tpu as pltpu
from jax.experimental.pallas import tpu_sc as plsc

N = 10000
NP = 10240
E = 160000
TILES = 16
CB = 128
NCHUNK = 79
EP = TILES * NCHUNK * CB
ROWS_PT = NP // TILES
CH = 64
BM = 512


def _fill(ref, rows, cols, val):
    v = jnp.full((16,), val, jnp.float32)

    def body(r, _):
        for c in range(cols // 16):
            ref[r, pl.ds(c * 16, 16)] = v
        return 0

    lax.fori_loop(0, rows, body, 0)


def _make_agg(nc, with_deg):
    out_type = [jax.ShapeDtypeStruct((NP, CH), jnp.float32) for _ in range(2 * nc)]
    if with_deg:
        out_type.append(jax.ShapeDtypeStruct((TILES, NP), jnp.float32))

    scratch = [
        pltpu.VMEM((NCHUNK, CB), jnp.int32),
        pltpu.VMEM((NCHUNK, CB), jnp.int32),
        pltpu.VMEM((4, CB, CH), jnp.float32),
        pltpu.VMEM((CB, CH), jnp.float32),
        pltpu.VMEM_SHARED((NP, CH), jnp.float32),
    ]
    if with_deg:
        scratch.append(pltpu.VMEM((NP,), jnp.float32))
    scratch.extend([pltpu.SemaphoreType.DMA] * 8)

    mesh = plsc.VectorSubcoreMesh(core_axis_name="c", subcore_axis_name="s")

    @functools.partial(pl.kernel, out_type=out_type, mesh=mesh,
                       scratch_types=scratch,
                       compiler_params=pltpu.CompilerParams(
                           needs_layout_passes=False,
                           use_tc_tiling_on_sc=False))
    def agg(src_hbm, dst_hbm, *rest):
        tables = rest[:2 * nc]
        outs = rest[2 * nc:4 * nc]
        k = 4 * nc
        if with_deg:
            deg_out = rest[k]; k += 1
        src_v = rest[k]; dst_v = rest[k + 1]
        rows_v = rest[k + 2]; zrow_v = rest[k + 3]
        acc = rest[k + 4]; k += 5
        if with_deg:
            hist_v = rest[k]; k += 1
        gsems = rest[k:k + 4]
        ssems = rest[k + 4:k + 8]

        core = lax.axis_index("c")
        sid = lax.axis_index("s")
        row0 = sid * ROWS_PT

        pltpu.sync_copy(src_hbm.at[sid], src_v)
        pltpu.sync_copy(dst_hbm.at[sid], dst_v)

        _fill(zrow_v, CB, CH, 0.0)
        if with_deg:
            @pl.when(core == 0)
            def _():
                zv = jnp.zeros((16,), jnp.float32)

                def zb(rr, _):
                    hist_v[pl.ds(rr * 16, 16)] = zv
                    return 0
                lax.fori_loop(0, NP // 16, zb, 0)

                ones16 = jnp.ones((16,), jnp.float32)

                def db(j, _):
                    for c16 in range(CB // 16):
                        idx = dst_v[j, pl.ds(c16 * 16, 16)]
                        plsc.addupdate_scatter(hist_v, [idx], ones16)
                    return 0
                lax.fori_loop(0, NCHUNK, db, 0)
                pltpu.sync_copy(hist_v, deg_out.at[sid])

        for i in range(nc):
            for b in range(ROWS_PT // CB):
                pltpu.sync_copy(zrow_v, acc.at[pl.ds(row0 + b * CB, CB)])
            plsc.subcore_barrier()

            for c in range(2):
                @pl.when(core == c)
                def _(c=c, i=i):
                    tab = tables[c * nc + i]

                    def g_issue(j, p):
                        pltpu.async_copy(tab.at[src_v.at[j]], rows_v.at[p],
                                         gsems[p])

                    def g_wait(j, p):
                        pltpu.make_async_copy(tab.at[src_v.at[j]],
                                              rows_v.at[p], gsems[p]).wait()

                    def s_issue(j, p):
                        pltpu.async_copy(rows_v.at[p], acc.at[dst_v.at[j]],
                                         ssems[p], add=True)

                    def s_wait(j, p):
                        pltpu.make_async_copy(rows_v.at[p],
                                              acc.at[dst_v.at[j]],
                                              ssems[p]).wait()

                    for p in range(4):
                        g_issue(p, p)

                    def edge_body(t4, _):
                        j = 4 * t4
                        g_wait(j, 0); s_issue(j, 0)
                        g_wait(j + 1, 1); s_issue(j + 1, 1)
                        s_wait(j, 0); g_issue(j + 4, 0)
                        g_wait(j + 2, 2); s_issue(j + 2, 2)
                        s_wait(j + 1, 1); g_issue(j + 5, 1)
                        g_wait(j + 3, 3); s_issue(j + 3, 3)
                        s_wait(j + 2, 2); g_issue(j + 6, 2)
                        s_wait(j + 3, 3)

                        @pl.when(j + 7 < NCHUNK)
                        def _():
                            g_issue(j + 7, 3)
                        return 0

                    lax.fori_loop(0, NCHUNK // 4, edge_body, 0)
                    for p in range(NCHUNK % 4):
                        j = NCHUNK - NCHUNK % 4 + p
                        g_wait(j, p); s_issue(j, p)
                    for p in range(NCHUNK % 4):
                        j = NCHUNK - NCHUNK % 4 + p
                        s_wait(j, p)
            plsc.subcore_barrier()

            for c in range(2):
                @pl.when(core == c)
                def _(c=c, i=i):
                    pltpu.sync_copy(acc.at[pl.ds(row0, ROWS_PT)],
                                    outs[c * nc + i].at[pl.ds(row0, ROWS_PT)])

    return agg


_agg1 = _make_agg(nc=2, with_deg=True)
_agg2 = _make_agg(nc=4, with_deg=False)


def _tc1_body(x_ref, a0_ref, a1_ref, a2_ref, a3_ref, d_ref, ws_ref, wn_ref,
              b_ref, *o_refs):
    deg = jnp.sum(d_ref[...], axis=0).reshape(BM, 1)
    r = 1.0 / jnp.maximum(deg, 1.0)
    agg = jnp.concatenate(
        [a0_ref[...], a1_ref[...], a2_ref[...], a3_ref[...]], axis=1) * r
    h = jnp.dot(x_ref[...], ws_ref[...], preferred_element_type=jnp.float32)
    h = h + jnp.dot(agg, wn_ref[...], preferred_element_type=jnp.float32)
    h = jnp.maximum(h + b_ref[...], 0.0)
    for c in range(8):
        o_refs[c][...] = h[:, c * CH:(c + 1) * CH]


_tc1 = pl.pallas_call(
    _tc1_body,
    grid=(NP // BM,),
    in_specs=[
        pl.BlockSpec((BM, 256), lambda m: (m, 0)),
        pl.BlockSpec((BM, CH), lambda m: (m, 0)),
        pl.BlockSpec((BM, CH), lambda m: (m, 0)),
        pl.BlockSpec((BM, CH), lambda m: (m, 0)),
        pl.BlockSpec((BM, CH), lambda m: (m, 0)),
        pl.BlockSpec((TILES, BM), lambda m: (0, m)),
        pl.BlockSpec((256, 512), lambda m: (0, 0)),
        pl.BlockSpec((256, 512), lambda m: (0, 0)),
        pl.BlockSpec((1, 512), lambda m: (0, 0)),
    ],
    out_specs=[pl.BlockSpec((BM, CH), lambda m: (m, 0)) for _ in range(8)],
    out_shape=[jax.ShapeDtypeStruct((NP, CH), jnp.float32) for _ in range(8)],
)


def _tc2_body(h0_ref, h1_ref, h2_ref, h3_ref, h4_ref, h5_ref, h6_ref, h7_ref,
              g0_ref, g1_ref, g2_ref, g3_ref, g4_ref, g5_ref, g6_ref,
              g7_ref, d_ref, ws_ref, wn_ref,
              b1_ref, rw0_ref, rb0_ref, rw1_ref, rb1_ref, rw2_ref, rb2_ref,
              o_ref):
    deg = jnp.sum(d_ref[...], axis=0).reshape(BM, 1)
    r = 1.0 / jnp.maximum(deg, 1.0)
    h = jnp.concatenate(
        [h0_ref[...], h1_ref[...], h2_ref[...], h3_ref[...],
         h4_ref[...], h5_ref[...], h6_ref[...], h7_ref[...]], axis=1)
    neigh = jnp.concatenate(
        [g0_ref[...], g1_ref[...], g2_ref[...], g3_ref[...],
         g4_ref[...], g5_ref[...], g6_ref[...], g7_ref[...]], axis=1) * r
    h2 = jnp.dot(h, ws_ref[...], preferred_element_type=jnp.float32)
    h2 = h2 + jnp.dot(neigh, wn_ref[...], preferred_element_type=jnp.float32)
    h2 = jnp.maximum(h2 + b1_ref[...], 0.0)
    t = jnp.maximum(
        jnp.dot(h2, rw0_ref[...], preferred_element_type=jnp.float32)
        + rb0_ref[...], 0.0)
    t = jnp.maximum(
        jnp.dot(t, rw1_ref[...], preferred_element_type=jnp.float32)
        + rb1_ref[...], 0.0)
    o_ref[...] = (jnp.dot(t, rw2_ref[...], preferred_element_type=jnp.float32)
                  + rb2_ref[...])


_tc2 = pl.pallas_call(
    _tc2_body,
    grid=(NP // BM,),
    in_specs=(
        [pl.BlockSpec((BM, CH), lambda m: (m, 0)) for _ in range(16)]
        + [
            pl.BlockSpec((TILES, BM), lambda m: (0, m)),
            pl.BlockSpec((512, 512), lambda m: (0, 0)),
            pl.BlockSpec((512, 512), lambda m: (0, 0)),
            pl.BlockSpec((1, 512), lambda m: (0, 0)),
            pl.BlockSpec((512, 512), lambda m: (0, 0)),
            pl.BlockSpec((1, 512), lambda m: (0, 0)),
            pl.BlockSpec((512, 256), lambda m: (0, 0)),
            pl.BlockSpec((1, 256), lambda m: (0, 0)),
            pl.BlockSpec((256, 1), lambda m: (0, 0)),
            pl.BlockSpec((1, 1), lambda m: (0, 0)),
        ]
    ),
    out_specs=pl.BlockSpec((BM, 1), lambda m: (m, 0)),
    out_shape=jax.ShapeDtypeStruct((NP, 1), jnp.float32),
)


def kernel(x, edge_index, W_self0, W_neigh0, b0, W_self1, W_neigh1, b1,
           R_W0, R_b0, R_W1, R_b1, R_W2, R_b2):
    f32 = jnp.float32
    src = edge_index[0]
    dst = edge_index[1]
    padv = jnp.full((EP - E,), NP - 1, jnp.int32)
    src_p = jnp.concatenate([src, padv]).reshape(TILES, NCHUNK, CB)
    dst_p = jnp.concatenate([dst, padv]).reshape(TILES, NCHUNK, CB)
    xp = jnp.pad(x, ((0, NP - N), (0, 0)))

    xs = [lax.slice(xp, (0, c * CH), (NP, (c + 1) * CH)) for c in range(4)]
    a0, a1, a2, a3, degp = _agg1(src_p, dst_p, *xs)
    hs = _tc1(xp, a0, a1, a2, a3, degp, W_self0, W_neigh0,
              b0.reshape(1, -1))
    gs = _agg2(src_p, dst_p, *hs)
    out = _tc2(*hs, *gs, degp, W_self1, W_neigh1,
               b1.reshape(1, -1), R_W0, R_b0.reshape(1, -1),
               R_W1, R_b1.reshape(1, -1), R_W2, R_b2.reshape(1, -1))
    return out[:N]
